# baseline (device time: 30020 ns/iter reference)
import jax
import jax.numpy as jnp
from jax import lax
from jax.experimental import pallas as pl
from jax.experimental.pallas import tpu as pltpu

K = 4


def kernel(x):
    m, n = x.shape
    nh = n // 2
    blk = m // K

    def body(x_ref, out_ref, ysend_sems, yrecv_sems, copy_sem):
        my_x = lax.axis_index("x")
        my_y = lax.axis_index("y")
        my_z = lax.axis_index("z")
        other_y = 1 - my_y
        ypeer = (my_x, other_y, my_z)

        barrier = pltpu.get_barrier_semaphore()
        pl.semaphore_signal(
            barrier, inc=1, device_id=ypeer,
            device_id_type=pl.DeviceIdType.MESH,
        )
        pl.semaphore_wait(barrier, 1)

        dst_row0 = my_y * m
        y_rdmas = []
        for k in range(K):
            r = pltpu.make_async_remote_copy(
                src_ref=x_ref.at[
                    pl.ds(k * blk, blk), pl.ds(other_y * nh, nh)
                ],
                dst_ref=out_ref.at[pl.ds(dst_row0 + k * blk, blk), :],
                send_sem=ysend_sems.at[k],
                recv_sem=yrecv_sems.at[k],
                device_id=ypeer,
                device_id_type=pl.DeviceIdType.MESH,
            )
            r.start()
            y_rdmas.append(r)

        local_copy = pltpu.make_async_copy(
            x_ref.at[:, pl.ds(my_y * nh, nh)],
            out_ref.at[pl.ds(my_y * m, m), :],
            copy_sem,
        )
        local_copy.start()

        for k in range(K):
            y_rdmas[k].wait_send()
            y_rdmas[k].wait_recv()
        local_copy.wait()

    return pl.pallas_call(
        body,
        out_shape=jax.ShapeDtypeStruct((2 * m, nh), x.dtype),
        in_specs=[pl.BlockSpec(memory_space=pltpu.VMEM)],
        out_specs=pl.BlockSpec(memory_space=pltpu.VMEM),
        scratch_shapes=[
            pltpu.SemaphoreType.DMA((K,)),
            pltpu.SemaphoreType.DMA((K,)),
            pltpu.SemaphoreType.DMA,
        ],
        compiler_params=pltpu.CompilerParams(collective_id=0),
    )(x)


# device time: 22950 ns/iter; 1.3081x vs baseline; 1.3081x over previous
import jax
import jax.numpy as jnp
from jax import lax
from jax.experimental import pallas as pl
from jax.experimental.pallas import tpu as pltpu

K = 8


def kernel(x):
    m, n = x.shape
    nh = n // 2
    half = m // 2
    blk = half // K

    def body(x_ref, out_ref, ysend_sems, yrecv_sems, xsend_sems, xrecv_sems,
             copy_sem):
        my_x = lax.axis_index("x")
        my_y = lax.axis_index("y")
        my_z = lax.axis_index("z")
        other_y = 1 - my_y
        other_x = 1 - my_x
        ypeer = (my_x, other_y, my_z)
        xpeer = (other_x, my_y, my_z)

        barrier = pltpu.get_barrier_semaphore()
        for p in (ypeer, xpeer):
            pl.semaphore_signal(
                barrier, inc=1, device_id=p,
                device_id_type=pl.DeviceIdType.MESH,
            )
        pl.semaphore_wait(barrier, 2)

        src_row0 = my_x * half
        dst_row0 = my_y * m + my_x * half
        fwd_row0 = other_y * m + my_x * half

        y_rdmas = []
        for k in range(K):
            r = pltpu.make_async_remote_copy(
                src_ref=x_ref.at[
                    pl.ds(src_row0 + k * blk, blk), pl.ds(other_y * nh, nh)
                ],
                dst_ref=out_ref.at[pl.ds(dst_row0 + k * blk, blk), :],
                send_sem=ysend_sems.at[k],
                recv_sem=yrecv_sems.at[k],
                device_id=ypeer,
                device_id_type=pl.DeviceIdType.MESH,
            )
            r.start()
            y_rdmas.append(r)

        local_copy = pltpu.make_async_copy(
            x_ref.at[:, pl.ds(my_y * nh, nh)],
            out_ref.at[pl.ds(my_y * m, m), :],
            copy_sem,
        )
        local_copy.start()

        x_rdmas = []
        for k in range(K):
            y_rdmas[k].wait_recv()
            r = pltpu.make_async_remote_copy(
                src_ref=out_ref.at[pl.ds(fwd_row0 + k * blk, blk), :],
                dst_ref=out_ref.at[pl.ds(fwd_row0 + k * blk, blk), :],
                send_sem=xsend_sems.at[k],
                recv_sem=xrecv_sems.at[k],
                device_id=xpeer,
                device_id_type=pl.DeviceIdType.MESH,
            )
            r.start()
            x_rdmas.append(r)

        for k in range(K):
            y_rdmas[k].wait_send()
            x_rdmas[k].wait_send()
            x_rdmas[k].wait_recv()
        local_copy.wait()

    return pl.pallas_call(
        body,
        out_shape=jax.ShapeDtypeStruct((2 * m, nh), x.dtype),
        in_specs=[pl.BlockSpec(memory_space=pltpu.MemorySpace.HBM)],
        out_specs=pl.BlockSpec(memory_space=pltpu.MemorySpace.HBM),
        scratch_shapes=[
            pltpu.SemaphoreType.DMA((K,)),
            pltpu.SemaphoreType.DMA((K,)),
            pltpu.SemaphoreType.DMA((K,)),
            pltpu.SemaphoreType.DMA((K,)),
            pltpu.SemaphoreType.DMA,
        ],
        compiler_params=pltpu.CompilerParams(collective_id=0),
    )(x)
